# baseline (speedup 1.0000x reference)
import functools

import jax
import jax.numpy as jnp
from jax.experimental import pallas as pl
from jax.experimental.pallas import tpu as pltpu

LATENT = 32
BATCH = 16384
STEP = 128
NSTEP = BATCH // STEP


def _tc_body(uid_s, iid_s, uf_hbm, if_hbm, o_ref, ubuf, ibuf, sem_u, sem_i):
    step = pl.program_id(0)

    def issue(c, slot):
        for k in range(STEP):
            ru = uid_s[c * STEP + k]
            ri = iid_s[c * STEP + k]
            pltpu.make_async_copy(
                uf_hbm.at[pl.ds(ru, 1)], ubuf.at[slot].at[pl.ds(k, 1)],
                sem_u.at[slot]).start()
            pltpu.make_async_copy(
                if_hbm.at[pl.ds(ri, 1)], ibuf.at[slot].at[pl.ds(k, 1)],
                sem_i.at[slot]).start()

    def drain(slot):
        pltpu.make_async_copy(
            uf_hbm.at[pl.ds(0, STEP)], ubuf.at[slot], sem_u.at[slot]).wait()
        pltpu.make_async_copy(
            if_hbm.at[pl.ds(0, STEP)], ibuf.at[slot], sem_i.at[slot]).wait()

    @pl.when(step == 0)
    def _():
        issue(0, 0)

    @pl.when(step + 1 < NSTEP)
    def _():
        issue(step + 1, (step + 1) % 2)

    slot = step % 2
    drain(slot)
    u = ubuf[slot]
    i = ibuf[slot]
    o_ref[...] = jnp.sum(u * i, axis=1)


def kernel(user_ids, item_ids, user_factors, item_factors):
    grid_spec = pltpu.PrefetchScalarGridSpec(
        num_scalar_prefetch=2,
        grid=(NSTEP,),
        in_specs=[
            pl.BlockSpec(memory_space=pltpu.MemorySpace.HBM),
            pl.BlockSpec(memory_space=pltpu.MemorySpace.HBM),
        ],
        out_specs=pl.BlockSpec((STEP,), lambda i, *_: (i,)),
        scratch_shapes=[
            pltpu.VMEM((2, STEP, LATENT), jnp.float32),
            pltpu.VMEM((2, STEP, LATENT), jnp.float32),
            pltpu.SemaphoreType.DMA((2,)),
            pltpu.SemaphoreType.DMA((2,)),
        ],
    )
    return pl.pallas_call(
        _tc_body,
        grid_spec=grid_spec,
        out_shape=jax.ShapeDtypeStruct((BATCH,), jnp.float32),
    )(user_ids, item_ids, user_factors, item_factors)


# speedup vs baseline: 1.0739x; 1.0739x over previous
import functools

import jax
import jax.numpy as jnp
from jax import lax
from jax.experimental import pallas as pl
from jax.experimental.pallas import tpu as pltpu
from jax.experimental.pallas import tpu_sc as plsc

LATENT = 32
BATCH = 16384
NC = 2
NS = 16
NW = NC * NS
CHUNK = 128

SC_N = 8192
TC_N = BATCH - SC_N
B_PER_W = SC_N // NW
NCHUNK = B_PER_W // CHUNK
STEP = 128
NSTEP = TC_N // STEP


def _sc_body(uid_hbm, iid_hbm, uf_hbm, if_hbm, out_hbm,
             uid_v, iid_v, ubuf0, ubuf1, ibuf0, ibuf1, out_v,
             sem_u, sem_i, sem_out):
    wid = lax.axis_index("s") * NC + lax.axis_index("c")
    base = wid * B_PER_W

    pltpu.sync_copy(uid_hbm.at[pl.ds(base, B_PER_W)], uid_v)
    pltpu.sync_copy(iid_hbm.at[pl.ds(base, B_PER_W)], iid_v)

    lane = lax.iota(jnp.int32, 16)
    ubufs = (ubuf0, ubuf1)
    ibufs = (ibuf0, ibuf1)

    def issue(c, slot):
        ub = ubufs[slot]
        ib = ibufs[slot]

        def grp(g, _):
            iu = uid_v[pl.ds(c * CHUNK + g * 16, 16)]
            ii = iid_v[pl.ds(c * CHUNK + g * 16, 16)]
            for k in range(16):
                b = g * 16 + k
                pltpu.async_copy(uf_hbm.at[pl.ds(iu[k], 1)],
                                 ub.at[pl.ds(b, 1)], sem_u)
                pltpu.async_copy(if_hbm.at[pl.ds(ii[k], 1)],
                                 ib.at[pl.ds(b, 1)], sem_i)
            return 0

        lax.fori_loop(0, CHUNK // 16, grp, 0)

    def drain(sem, buf):
        pltpu.make_async_copy(uf_hbm.at[pl.ds(0, CHUNK)], buf, sem).wait()

    def compute(c, slot):
        ub = ubufs[slot]
        ib = ibufs[slot]

        def group(g, _):
            rows = g * 16 + lane
            acc = jnp.zeros((16,), jnp.float32)
            for d in range(LATENT):
                col = jnp.full((16,), d, jnp.int32)
                uc = plsc.load_gather(ub, [rows, col])
                ic = plsc.load_gather(ib, [rows, col])
                acc = acc + uc * ic
            out_v[pl.ds(c * CHUNK + g * 16, 16)] = acc
            return 0

        lax.fori_loop(0, CHUNK // 16, group, 0)

    issue(0, 0)
    for c in range(NCHUNK):
        if c + 1 < NCHUNK:
            issue(c + 1, (c + 1) % 2)
        drain(sem_u, ubufs[c % 2])
        drain(sem_i, ibufs[c % 2])
        compute(c, c % 2)

    pltpu.async_copy(out_v, out_hbm.at[pl.ds(base, B_PER_W)], sem_out).wait()


def _sc_call(uid, iid, uf, if_):
    mesh = plsc.VectorSubcoreMesh(core_axis_name="c", subcore_axis_name="s")
    run = functools.partial(
        pl.kernel, mesh=mesh,
        out_type=jax.ShapeDtypeStruct((SC_N,), jnp.float32),
        compiler_params=pltpu.CompilerParams(needs_layout_passes=False),
        scratch_types=[
            pltpu.VMEM((B_PER_W,), jnp.int32),
            pltpu.VMEM((B_PER_W,), jnp.int32),
            pltpu.VMEM((CHUNK, LATENT), jnp.float32),
            pltpu.VMEM((CHUNK, LATENT), jnp.float32),
            pltpu.VMEM((CHUNK, LATENT), jnp.float32),
            pltpu.VMEM((CHUNK, LATENT), jnp.float32),
            pltpu.VMEM((B_PER_W,), jnp.float32),
            pltpu.SemaphoreType.DMA,
            pltpu.SemaphoreType.DMA,
            pltpu.SemaphoreType.DMA,
        ],
    )(_sc_body)
    return run(uid, iid, uf, if_)


def _tc_body(uid_s, iid_s, uf_hbm, if_hbm, o_ref, ubuf, ibuf, sem_u, sem_i):
    step = pl.program_id(0)

    def issue(c, slot):
        for k in range(STEP):
            ru = uid_s[c * STEP + k]
            ri = iid_s[c * STEP + k]
            pltpu.make_async_copy(
                uf_hbm.at[pl.ds(ru, 1)], ubuf.at[slot].at[pl.ds(k, 1)],
                sem_u.at[slot]).start()
            pltpu.make_async_copy(
                if_hbm.at[pl.ds(ri, 1)], ibuf.at[slot].at[pl.ds(k, 1)],
                sem_i.at[slot]).start()

    def drain(slot):
        pltpu.make_async_copy(
            uf_hbm.at[pl.ds(0, STEP)], ubuf.at[slot], sem_u.at[slot]).wait()
        pltpu.make_async_copy(
            if_hbm.at[pl.ds(0, STEP)], ibuf.at[slot], sem_i.at[slot]).wait()

    @pl.when(step == 0)
    def _():
        issue(0, 0)

    @pl.when(step + 1 < NSTEP)
    def _():
        issue(step + 1, (step + 1) % 2)

    slot = step % 2
    drain(slot)
    u = ubuf[slot]
    i = ibuf[slot]
    o_ref[...] = jnp.sum(u * i, axis=1)


def _tc_call(uid, iid, uf, if_):
    grid_spec = pltpu.PrefetchScalarGridSpec(
        num_scalar_prefetch=2,
        grid=(NSTEP,),
        in_specs=[
            pl.BlockSpec(memory_space=pltpu.MemorySpace.HBM),
            pl.BlockSpec(memory_space=pltpu.MemorySpace.HBM),
        ],
        out_specs=pl.BlockSpec((STEP,), lambda i, *_: (i,)),
        scratch_shapes=[
            pltpu.VMEM((2, STEP, LATENT), jnp.float32),
            pltpu.VMEM((2, STEP, LATENT), jnp.float32),
            pltpu.SemaphoreType.DMA((2,)),
            pltpu.SemaphoreType.DMA((2,)),
        ],
    )
    return pl.pallas_call(
        _tc_body,
        grid_spec=grid_spec,
        out_shape=jax.ShapeDtypeStruct((TC_N,), jnp.float32),
    )(uid, iid, uf, if_)


def kernel(user_ids, item_ids, user_factors, item_factors):
    out_sc = _sc_call(user_ids[:SC_N], item_ids[:SC_N],
                      user_factors, item_factors)
    out_tc = _tc_call(user_ids[SC_N:], item_ids[SC_N:],
                      user_factors, item_factors)
    return jnp.concatenate([out_sc, out_tc])


# speedup vs baseline: 1.1570x; 1.0773x over previous
import functools

import jax
import jax.numpy as jnp
from jax import lax
from jax.experimental import pallas as pl
from jax.experimental.pallas import tpu as pltpu
from jax.experimental.pallas import tpu_sc as plsc

LATENT = 32
BATCH = 16384
NC = 2
NS = 16
NW = NC * NS
B_PER_W = BATCH // NW
CHUNK = 128
NCHUNK = B_PER_W // CHUNK


def _nmf_body(uid_hbm, iid_hbm, uf_hbm, if_hbm, out_hbm,
              uid_v, iid_v, ubuf0, ubuf1, ibuf0, ibuf1, out_v,
              sem_u, sem_i, sem_out):
    wid = lax.axis_index("s") * NC + lax.axis_index("c")
    base = wid * B_PER_W

    pltpu.sync_copy(uid_hbm.at[pl.ds(base, B_PER_W)], uid_v)
    pltpu.sync_copy(iid_hbm.at[pl.ds(base, B_PER_W)], iid_v)

    lane = lax.iota(jnp.int32, 16)
    ubufs = (ubuf0, ubuf1)
    ibufs = (ibuf0, ibuf1)

    def issue(c, slot):
        ub = ubufs[slot]
        ib = ibufs[slot]

        def grp(g, _):
            iu = uid_v[pl.ds(c * CHUNK + g * 16, 16)]
            ii = iid_v[pl.ds(c * CHUNK + g * 16, 16)]
            for k in range(16):
                b = g * 16 + k
                pltpu.async_copy(uf_hbm.at[pl.ds(iu[k], 1)],
                                 ub.at[pl.ds(b, 1)], sem_u)
                pltpu.async_copy(if_hbm.at[pl.ds(ii[k], 1)],
                                 ib.at[pl.ds(b, 1)], sem_i)
            return 0

        lax.fori_loop(0, CHUNK // 16, grp, 0)

    def drain(sem, buf):
        pltpu.make_async_copy(uf_hbm.at[pl.ds(0, CHUNK)], buf, sem).wait()

    def compute(c, slot):
        ub = ubufs[slot]
        ib = ibufs[slot]

        def group(g, _):
            rows = g * 16 + lane
            acc = jnp.zeros((16,), jnp.float32)
            for d in range(LATENT):
                col = jnp.full((16,), d, jnp.int32)
                uc = plsc.load_gather(ub, [rows, col])
                ic = plsc.load_gather(ib, [rows, col])
                acc = acc + uc * ic
            out_v[pl.ds(c * CHUNK + g * 16, 16)] = acc
            return 0

        lax.fori_loop(0, CHUNK // 16, group, 0)

    issue(0, 0)
    for c in range(NCHUNK):
        if c + 1 < NCHUNK:
            issue(c + 1, (c + 1) % 2)
        drain(sem_u, ubufs[c % 2])
        drain(sem_i, ibufs[c % 2])
        compute(c, c % 2)

    pltpu.async_copy(out_v, out_hbm.at[pl.ds(base, B_PER_W)], sem_out).wait()


def kernel(user_ids, item_ids, user_factors, item_factors):
    mesh = plsc.VectorSubcoreMesh(core_axis_name="c", subcore_axis_name="s")
    run = functools.partial(
        pl.kernel, mesh=mesh,
        out_type=jax.ShapeDtypeStruct((BATCH,), jnp.float32),
        compiler_params=pltpu.CompilerParams(needs_layout_passes=False),
        scratch_types=[
            pltpu.VMEM((B_PER_W,), jnp.int32),
            pltpu.VMEM((B_PER_W,), jnp.int32),
            pltpu.VMEM((CHUNK, LATENT), jnp.float32),
            pltpu.VMEM((CHUNK, LATENT), jnp.float32),
            pltpu.VMEM((CHUNK, LATENT), jnp.float32),
            pltpu.VMEM((CHUNK, LATENT), jnp.float32),
            pltpu.VMEM((B_PER_W,), jnp.float32),
            pltpu.SemaphoreType.DMA,
            pltpu.SemaphoreType.DMA,
            pltpu.SemaphoreType.DMA,
        ],
    )(_nmf_body)
    return run(user_ids, item_ids, user_factors, item_factors)
